# baseline (speedup 1.0000x reference)
import functools

import jax
import jax.numpy as jnp
from jax import lax
from jax.experimental import pallas as pl
from jax.experimental.pallas import tpu as pltpu
from jax.experimental.pallas import tpu_sc as plsc

B, N, C = 4, 4096, 16
BN = B * N
EG = 32768
E = B * EG
M1 = 256
_NEG = -1e30
_SC_PARAMS = pltpu.CompilerParams(use_tc_tiling_on_sc=False)


_MESH = dict(core_axis_name="c", subcore_axis_name="s")


def _sc_deg(row_sc):
    @functools.partial(
        pl.kernel,
        mesh=plsc.VectorSubcoreMesh(**_MESH),
        compiler_params=_SC_PARAMS,
        out_type=jax.ShapeDtypeStruct((2048, 128), jnp.float32),
        scratch_types=[
            pltpu.VMEM((32, 128), jnp.int32),
            pltpu.VMEM((128, 16), jnp.float32),
            pltpu.VMEM((512, 16), jnp.float32),
            pltpu.VMEM((64, 128), jnp.float32),
            pltpu.VMEM_SHARED((8192, 16), jnp.float32),
            pltpu.SemaphoreType.DMA,
        ],
    )
    def k(row_hbm, deg_hbm, idx_v, ones_v, v16, v128, acc_sh, sem):
        c = lax.axis_index("c")
        s = lax.axis_index("s")
        pltpu.sync_copy(row_hbm.at[pl.ds((c * 16 + s) * 32, 32)], idx_v)

        def fill(j, carry):
            v16[j, :] = jnp.zeros((16,), jnp.float32)
            ones_v[lax.rem(j, 128), :] = jnp.full((16,), 1.0, jnp.float32)
            return carry

        lax.fori_loop(0, 512, fill, 0)
        pltpu.sync_copy(v16, acc_sh.at[pl.ds(s * 512, 512)])
        plsc.subcore_barrier()

        def grp(g, carry):
            for b in range(8):
                pltpu.async_copy(ones_v, acc_sh.at[idx_v.at[g * 8 + b]], sem,
                                 add=True)
            for b in range(8):
                pltpu.make_async_copy(ones_v, acc_sh.at[idx_v.at[0]],
                                      sem).wait()
            return carry

        lax.fori_loop(0, 4, grp, 0)
        plsc.subcore_barrier()
        pltpu.sync_copy(acc_sh.at[pl.ds(s * 512, 512)], v16)

        def rp(j, carry):
            for kk in range(8):
                v128[j, pl.ds(kk * 16, 16)] = v16[j * 8 + kk, :]
            return carry

        lax.fori_loop(0, 64, rp, 0)
        pltpu.sync_copy(v128, deg_hbm.at[pl.ds(c * 1024 + s * 64, 64)])

    return k(row_sc)


def _sc_edge_accum(gather_idx, scatter_idx, tables, zeros, rows_per_sc,
                   chunks_per_tile, n_phase, rpc=2, w=128, nbuf=2):
    stripe = rows_per_sc // 16
    nt = len(tables)
    nch = chunks_per_tile

    @functools.partial(
        pl.kernel,
        mesh=plsc.VectorSubcoreMesh(**_MESH),
        compiler_params=_SC_PARAMS,
        out_type=[jax.ShapeDtypeStruct((n_phase * 2 * rows_per_sc, w),
                                       jnp.float32) for _ in range(nt)],
        scratch_types=[pltpu.VMEM((nch * rpc, 128), jnp.int32),
                       pltpu.VMEM((nch * rpc, 128), jnp.int32)]
        + [pltpu.VMEM((128 * rpc, w), jnp.float32)
           for _ in range(nbuf * nt)]
        + [pltpu.VMEM_SHARED((rows_per_sc, w), jnp.float32)
           for _ in range(nt)]
        + [pltpu.SemaphoreType.DMA for _ in range(2 * nbuf * nt)],
    )
    def k(gidx_hbm, sidx_hbm, *rest):
        tab_hbm = rest[:nt]
        zeros_hbm = rest[nt]
        out_hbm = rest[nt + 1:2 * nt + 1]
        sc = rest[2 * nt + 1:]
        gi_v, si_v = sc[0], sc[1]
        rows_v = [[sc[2 + nbuf * t + b] for b in range(nbuf)]
                  for t in range(nt)]
        o = 2 + nbuf * nt
        acc_sh = sc[o:o + nt]
        sem_g = [[sc[o + nt + nbuf * t + b] for b in range(nbuf)]
                 for t in range(nt)]
        sem_s = [[sc[o + nt + nbuf * nt + nbuf * t + b] for b in range(nbuf)]
                 for t in range(nt)]
        c = lax.axis_index("c")
        s = lax.axis_index("s")
        for p in range(n_phase):
            crow = ((n_phase * c + p) * 16 + s) * nch * rpc
            pltpu.sync_copy(gidx_hbm.at[pl.ds(crow, nch * rpc)], gi_v)
            pltpu.sync_copy(sidx_hbm.at[pl.ds(crow, nch * rpc)], si_v)
            for t in range(nt):
                pltpu.sync_copy(zeros_hbm,
                                acc_sh[t].at[pl.ds(s * stripe, stripe)])
            plsc.subcore_barrier()

            def pair(gp, carry):
                for b in range(nbuf):
                    i = gp * nbuf + b

                    @pl.when(i >= nbuf)
                    def _():
                        for t in range(nt):
                            for r in range(rpc):
                                pltpu.make_async_copy(
                                    rows_v[t][b].at[pl.ds(r * 128, 128)],
                                    acc_sh[t].at[si_v.at[0]],
                                    sem_s[t][b],
                                ).wait()

                    for t in range(nt):
                        for r in range(rpc):
                            pltpu.async_copy(
                                tab_hbm[t].at[gi_v.at[i * rpc + r]],
                                rows_v[t][b].at[pl.ds(r * 128, 128)],
                                sem_g[t][b])
                    for t in range(nt):
                        for r in range(rpc):
                            pltpu.make_async_copy(
                                tab_hbm[t].at[gi_v.at[0]],
                                rows_v[t][b].at[pl.ds(r * 128, 128)],
                                sem_g[t][b]).wait()
                        for r in range(rpc):
                            pltpu.async_copy(
                                rows_v[t][b].at[pl.ds(r * 128, 128)],
                                acc_sh[t].at[si_v.at[i * rpc + r]],
                                sem_s[t][b], add=True)
                return carry

            lax.fori_loop(0, nch // nbuf, pair, 0)
            for b in range(nbuf):
                for t in range(nt):
                    for r in range(rpc):
                        pltpu.make_async_copy(
                            rows_v[t][b].at[pl.ds(r * 128, 128)],
                            acc_sh[t].at[si_v.at[0]],
                            sem_s[t][b]).wait()
            plsc.subcore_barrier()
            out_row = (n_phase * c + p) * rows_per_sc + s * stripe
            for t in range(nt):
                pltpu.sync_copy(
                    acc_sh[t].at[pl.ds(s * stripe, stripe)],
                    out_hbm[t].at[pl.ds(out_row, stripe)],
                )
            plsc.subcore_barrier()

    return k(gather_idx, scatter_idx, *tables, zeros)


def _tc_pre(xf, dis, W, b, Wr, br):
    def body(x_ref, d_ref, w_ref, b_ref, wr_ref, br_ref, hs_ref, y0_ref):
        x = x_ref[...]
        dis = jnp.max(d_ref[...], axis=-1, keepdims=True)
        h = jnp.dot(x, w_ref[...], preferred_element_type=jnp.float32) + b_ref[...]
        hs_ref[...] = dis * h
        y0_ref[...] = (
            jnp.dot(x, wr_ref[...], preferred_element_type=jnp.float32) + br_ref[...]
        )

    R = 2048
    grid = (BN // R,)
    blk = lambda r, cdim: pl.BlockSpec((r, cdim), lambda i: (i, 0))
    full = lambda a: pl.BlockSpec(a.shape, lambda i: (0,) * a.ndim)
    return pl.pallas_call(
        body,
        grid=grid,
        in_specs=[blk(R, 16), blk(R, 16), full(W), full(b), full(Wr), full(br)],
        out_specs=[blk(R, 64), blk(R, 64)],
        out_shape=[
            jax.ShapeDtypeStruct((BN, 64), jnp.float32),
            jax.ShapeDtypeStruct((BN, 64), jnp.float32),
        ],
    )(xf, dis, W, b, Wr, br)


def _tc_mid(acc, y0, dis, Wa, ba, Wec, bec):
    def body(a_ref, y_ref, d_ref, wa_ref, ba_ref, we_ref, be_ref,
             slo_ref, shi_ref, f_ref):
        dis = jnp.max(d_ref[...], axis=-1, keepdims=True)
        h1 = y_ref[...] + dis * a_ref[...]
        lg = jnp.dot(h1, wa_ref[...], preferred_element_type=jnp.float32) + ba_ref[...]
        lg = lg - jnp.max(lg, axis=-1, keepdims=True)
        ex = jnp.exp(lg)
        S = ex / jnp.sum(ex, axis=-1, keepdims=True)
        slo_ref[...] = S[:, :128]
        shi_ref[...] = S[:, 128:]
        f_ref[...] = (
            jnp.dot(h1, we_ref[...], preferred_element_type=jnp.float32) + be_ref[...]
        )

    R = 2048
    grid = (BN // R,)
    blk = lambda r, cdim: pl.BlockSpec((r, cdim), lambda i: (i, 0))
    full = lambda a: pl.BlockSpec(a.shape, lambda i: (0,) * a.ndim)
    return pl.pallas_call(
        body,
        grid=grid,
        in_specs=[blk(R, 64), blk(R, 64), blk(R, 16), full(Wa), full(ba),
                  full(Wec), full(bec)],
        out_specs=[blk(R, 128), blk(R, 128), blk(R, 64)],
        out_shape=[
            jax.ShapeDtypeStruct((BN, 128), jnp.float32),
            jax.ShapeDtypeStruct((BN, 128), jnp.float32),
            jax.ShapeDtypeStruct((BN, 64), jnp.float32),
        ],
    )(acc, y0, dis, Wa, ba, Wec, bec)


def _top4(A):
    M = A.shape[-1]
    lane = lax.broadcasted_iota(jnp.int32, A.shape, 1)
    cur = A
    idxs = []
    for _ in range(4):
        m = jnp.max(cur, axis=-1, keepdims=True)
        j = jnp.min(jnp.where(cur == m, lane, M), axis=-1, keepdims=True)
        idxs.append(j)
        cur = jnp.where(lane == j, _NEG, cur)
    return idxs


def _adj(idxs, M):
    lane = lax.broadcasted_iota(jnp.int32, (idxs[0].shape[0], M), 1)
    a = jnp.zeros((idxs[0].shape[0], M), jnp.float32)
    for j in idxs:
        a = a + (lane == j).astype(jnp.float32)
    return a


def _ct(a, b):
    return lax.dot_general(a, b, (((0,), (0,)), ((), ())),
                           preferred_element_type=jnp.float32)


def _gcn_small(x, Adj, W, b, Wr, br):
    h = jnp.dot(x, W, preferred_element_type=jnp.float32) + b
    y = jnp.dot(x, Wr, preferred_element_type=jnp.float32) + br
    return y + 0.25 * _ct(Adj, h)


def _pool_small(x, Adj, Wa, ba, Wec, bec, M, need_topk=True):
    lg = jnp.dot(x, Wa, preferred_element_type=jnp.float32) + ba
    lg = lg - jnp.max(lg, axis=-1, keepdims=True)
    ex = jnp.exp(lg)
    S = ex / jnp.sum(ex, axis=-1, keepdims=True)
    fcat = jnp.dot(x, Wec, preferred_element_type=jnp.float32) + bec
    Z = _ct(S, fcat)
    w = jnp.maximum(jnp.sum(S, axis=0, keepdims=True), 1e-10)
    col = lax.broadcasted_iota(jnp.int32, Z.shape, 1)
    xn = jnp.where(col < 3, Z / w.reshape(-1, 1), Z)
    if not need_topk:
        return xn, None
    A = _ct(S, jnp.dot(Adj, S, preferred_element_type=jnp.float32))
    return xn, _top4(A)


def _tc_tail(Slo, Shi, ASlo, AShi, fcat, TW):
    def body(slo_ref, shi_ref, alo_ref, ahi_ref, f_ref, *refs):
        w_refs = refs[:-2]
        xr_ref, z_ref = refs[-2:]
        P = {name: r[...] for name, r in zip(_TAIL_NAMES, w_refs)}
        Sg = jnp.concatenate([slo_ref[0], shi_ref[0]], axis=1)
        ASg = jnp.concatenate([alo_ref[0], ahi_ref[0]], axis=1)
        Z = _ct(Sg, f_ref[0])
        w = jnp.maximum(jnp.sum(Sg, axis=0, keepdims=True), 1e-10)
        col = lax.broadcasted_iota(jnp.int32, Z.shape, 1)
        x = jnp.where(col < 3, Z / w.reshape(-1, 1), Z)
        A = _ct(Sg, ASg)
        idxs = _top4(A)
        M = M1
        specs = [("enc_gcn2_", "enc_pool2_", 64), ("enc_gcn3_", "enc_pool3_", 16),
                 ("dec_gcn1_", "dec_pool1_", 64), ("dec_gcn2_", "dec_pool2_", 256),
                 ("dec_gcn3_", "dec_pool3_", 1024)]
        for i, (g, p, Mn) in enumerate(specs):
            Adj = _adj(idxs, M)
            x = _gcn_small(x, Adj, P[g + "W"], P[g + "b"], P[g + "Wr"],
                           P[g + "br"])
            last = i == len(specs) - 1
            x, idxs = _pool_small(x, Adj, P[p + "Wa"], P[p + "ba"],
                                  P[p + "Wec"], P[p + "bec"], Mn,
                                  need_topk=not last)
            M = Mn
            if p == "enc_pool3_":
                z_ref[0] = x
        hh = jnp.dot(x, P["head_W"], preferred_element_type=jnp.float32) + P["head_b"]
        lane = lax.broadcasted_iota(jnp.int32, hh.shape, 1)
        sig = 1.0 / (1.0 + jnp.exp(-hh))
        xr = jnp.where((lane >= 3) & (lane < 6), sig, hh)
        xr_ref[0] = xr[:1000]

    gblk = lambda shp: pl.BlockSpec((1,) + shp, lambda i: (i, 0, 0))
    full = lambda a: pl.BlockSpec(a.shape, lambda i: (0,) * a.ndim)
    return pl.pallas_call(
        body,
        grid=(B,),
        in_specs=[gblk((N, 128)), gblk((N, 128)), gblk((N, 128)),
                  gblk((N, 128)), gblk((N, 64))]
        + [full(TW[n]) for n in _TAIL_NAMES],
        out_specs=[gblk((1000, 16)), gblk((16, 256))],
        out_shape=[
            jax.ShapeDtypeStruct((B, 1000, 16), jnp.float32),
            jax.ShapeDtypeStruct((B, 16, 256), jnp.float32),
        ],
    )(Slo.reshape(B, N, 128), Shi.reshape(B, N, 128),
      ASlo.reshape(B, N, 128), AShi.reshape(B, N, 128),
      fcat.reshape(B, N, 64), *[TW[n] for n in _TAIL_NAMES])


_TAIL_NAMES = []
for _g, _p in [("enc_gcn2_", "enc_pool2_"), ("enc_gcn3_", "enc_pool3_"),
               ("dec_gcn1_", "dec_pool1_"), ("dec_gcn2_", "dec_pool2_"),
               ("dec_gcn3_", "dec_pool3_")]:
    _TAIL_NAMES += [_g + "W", _g + "b", _g + "Wr", _g + "br",
                    _p + "Wa", _p + "ba", _p + "Wec", _p + "bec"]
_TAIL_NAMES += ["head_W", "head_b"]


def _wecat(P, pre, Cin):
    Wec = jnp.concatenate([jnp.eye(Cin, dtype=jnp.float32)[:, :3],
                           P[pre + "We"]], axis=1)
    bec = jnp.concatenate([jnp.zeros(3, jnp.float32), P[pre + "be"]])
    return Wec, bec.reshape(1, -1)


def kernel(x, e_, params):
    P = params
    xf = x.reshape(BN, C)
    e_ = e_.astype(jnp.int32)
    row, col = e_[0], e_[1]
    goff = jnp.arange(B, dtype=jnp.int32)[:, None]
    half = (goff % 2) * N
    row_sc = (row[None] + half).reshape(-1, 128)
    row_g = (row[None] + goff * N).reshape(-1, 128)
    col_loc = jnp.tile(col, B).reshape(-1, 128)
    col_g = (col[None] + goff * N).reshape(-1, 128)
    row_loc = jnp.tile(row, B).reshape(-1, 128)

    zeros256 = jnp.zeros((256, 128), jnp.float32)
    zeros256_64 = jnp.zeros((256, 64), jnp.float32)

    deg = _sc_deg(row_sc).reshape(BN, 16)
    dis = jnp.where(deg > 0, deg ** -0.5, 0.0)

    hs, y0 = _tc_pre(xf, dis, P["enc_gcn1_W"], P["enc_gcn1_b"].reshape(1, -1),
                     P["enc_gcn1_Wr"], P["enc_gcn1_br"].reshape(1, -1))

    (acc,) = _sc_edge_accum(row_g, col_loc, [hs], zeros256_64, rows_per_sc=N,
                            chunks_per_tile=8, n_phase=2, w=64, nbuf=4)

    Wec1, bec1 = _wecat(P, "enc_pool1_", 64)
    Slo, Shi, fcat = _tc_mid(acc, y0, dis, P["enc_pool1_Wa"],
                             P["enc_pool1_ba"].reshape(1, -1), Wec1, bec1)

    (ASlo,) = _sc_edge_accum(col_g, row_loc, [Slo], zeros256,
                             rows_per_sc=N, chunks_per_tile=8, n_phase=2)
    (AShi,) = _sc_edge_accum(col_g, row_loc, [Shi], zeros256,
                             rows_per_sc=N, chunks_per_tile=8, n_phase=2)

    TW = {}
    for g, p in [("enc_gcn2_", "enc_pool2_"), ("enc_gcn3_", "enc_pool3_"),
                 ("dec_gcn1_", "dec_pool1_"), ("dec_gcn2_", "dec_pool2_"),
                 ("dec_gcn3_", "dec_pool3_")]:
        TW[g + "W"] = P[g + "W"]
        TW[g + "b"] = P[g + "b"].reshape(1, -1)
        TW[g + "Wr"] = P[g + "Wr"]
        TW[g + "br"] = P[g + "br"].reshape(1, -1)
        if p == "dec_pool3_":
            Wa = jnp.pad(P[p + "Wa"], ((0, 0), (0, 24)))
            ba = jnp.pad(P[p + "ba"], (0, 24), constant_values=_NEG)
            TW[p + "Wa"], TW[p + "ba"] = Wa, ba.reshape(1, -1)
        else:
            TW[p + "Wa"] = P[p + "Wa"]
            TW[p + "ba"] = P[p + "ba"].reshape(1, -1)
        TW[p + "Wec"], TW[p + "bec"] = _wecat(P, p, P[g + "W"].shape[1])
    TW["head_W"] = P["head_W"]
    TW["head_b"] = P["head_b"].reshape(1, -1)

    xr, z = _tc_tail(Slo, Shi, ASlo, AShi, fcat, TW)
    return xr, z.reshape(B * 16, 256)


# speedup vs baseline: 1.0013x; 1.0013x over previous
import functools

import jax
import jax.numpy as jnp
from jax import lax
from jax.experimental import pallas as pl
from jax.experimental.pallas import tpu as pltpu
from jax.experimental.pallas import tpu_sc as plsc

B, N, C = 4, 4096, 16
BN = B * N
EG = 32768
E = B * EG
M1 = 256
_NEG = -1e30
_SC_PARAMS = pltpu.CompilerParams(use_tc_tiling_on_sc=False)


_MESH = dict(core_axis_name="c", subcore_axis_name="s")


def _sc_deg(row_sc):
    @functools.partial(
        pl.kernel,
        mesh=plsc.VectorSubcoreMesh(**_MESH),
        compiler_params=_SC_PARAMS,
        out_type=jax.ShapeDtypeStruct((2048, 128), jnp.float32),
        scratch_types=[
            pltpu.VMEM((32, 128), jnp.int32),
            pltpu.VMEM((128, 16), jnp.float32),
            pltpu.VMEM((512, 16), jnp.float32),
            pltpu.VMEM((64, 128), jnp.float32),
            pltpu.VMEM_SHARED((8192, 16), jnp.float32),
            pltpu.SemaphoreType.DMA,
        ],
    )
    def k(row_hbm, deg_hbm, idx_v, ones_v, v16, v128, acc_sh, sem):
        c = lax.axis_index("c")
        s = lax.axis_index("s")
        pltpu.sync_copy(row_hbm.at[pl.ds((c * 16 + s) * 32, 32)], idx_v)

        def fill(j, carry):
            v16[j, :] = jnp.zeros((16,), jnp.float32)
            ones_v[lax.rem(j, 128), :] = jnp.full((16,), 1.0, jnp.float32)
            return carry

        lax.fori_loop(0, 512, fill, 0)
        pltpu.sync_copy(v16, acc_sh.at[pl.ds(s * 512, 512)])
        plsc.subcore_barrier()

        def grp(g, carry):
            for b in range(8):
                pltpu.async_copy(ones_v, acc_sh.at[idx_v.at[g * 8 + b]], sem,
                                 add=True)
            for b in range(8):
                pltpu.make_async_copy(ones_v, acc_sh.at[idx_v.at[0]],
                                      sem).wait()
            return carry

        lax.fori_loop(0, 4, grp, 0)
        plsc.subcore_barrier()
        pltpu.sync_copy(acc_sh.at[pl.ds(s * 512, 512)], v16)

        def rp(j, carry):
            for kk in range(8):
                v128[j, pl.ds(kk * 16, 16)] = v16[j * 8 + kk, :]
            return carry

        lax.fori_loop(0, 64, rp, 0)
        pltpu.sync_copy(v128, deg_hbm.at[pl.ds(c * 1024 + s * 64, 64)])

    return k(row_sc)


def _sc_edge_accum(gather_idx, scatter_idx, tables, zeros, rows_per_sc,
                   chunks_per_tile, n_phase, rpc=2, w=128, nbuf=2):
    stripe = rows_per_sc // 16
    nt = len(tables)
    nch = chunks_per_tile

    @functools.partial(
        pl.kernel,
        mesh=plsc.VectorSubcoreMesh(**_MESH),
        compiler_params=_SC_PARAMS,
        out_type=[jax.ShapeDtypeStruct((n_phase * 2 * rows_per_sc, w),
                                       jnp.float32) for _ in range(nt)],
        scratch_types=[pltpu.VMEM((nch * rpc, 128), jnp.int32),
                       pltpu.VMEM((nch * rpc, 128), jnp.int32)]
        + [pltpu.VMEM((128 * rpc, w), jnp.float32)
           for _ in range(nbuf * nt)]
        + [pltpu.VMEM_SHARED((rows_per_sc, w), jnp.float32)
           for _ in range(nt)]
        + [pltpu.SemaphoreType.DMA for _ in range(2 * nbuf * nt)],
    )
    def k(gidx_hbm, sidx_hbm, *rest):
        tab_hbm = rest[:nt]
        zeros_hbm = rest[nt]
        out_hbm = rest[nt + 1:2 * nt + 1]
        sc = rest[2 * nt + 1:]
        gi_v, si_v = sc[0], sc[1]
        rows_v = [[sc[2 + nbuf * t + b] for b in range(nbuf)]
                  for t in range(nt)]
        o = 2 + nbuf * nt
        acc_sh = sc[o:o + nt]
        sem_g = [[sc[o + nt + nbuf * t + b] for b in range(nbuf)]
                 for t in range(nt)]
        sem_s = [[sc[o + nt + nbuf * nt + nbuf * t + b] for b in range(nbuf)]
                 for t in range(nt)]
        c = lax.axis_index("c")
        s = lax.axis_index("s")
        for p in range(n_phase):
            crow = ((n_phase * c + p) * 16 + s) * nch * rpc
            pltpu.sync_copy(gidx_hbm.at[pl.ds(crow, nch * rpc)], gi_v)
            pltpu.sync_copy(sidx_hbm.at[pl.ds(crow, nch * rpc)], si_v)
            for t in range(nt):
                pltpu.sync_copy(zeros_hbm,
                                acc_sh[t].at[pl.ds(s * stripe, stripe)])
            plsc.subcore_barrier()

            def pair(gp, carry):
                for b in range(nbuf):
                    i = gp * nbuf + b

                    @pl.when(i >= nbuf)
                    def _():
                        for t in range(nt):
                            for r in range(rpc):
                                pltpu.make_async_copy(
                                    rows_v[t][b].at[pl.ds(r * 128, 128)],
                                    acc_sh[t].at[si_v.at[0]],
                                    sem_s[t][b],
                                ).wait()

                    for t in range(nt):
                        for r in range(rpc):
                            pltpu.async_copy(
                                tab_hbm[t].at[gi_v.at[i * rpc + r]],
                                rows_v[t][b].at[pl.ds(r * 128, 128)],
                                sem_g[t][b])
                    for t in range(nt):
                        for r in range(rpc):
                            pltpu.make_async_copy(
                                tab_hbm[t].at[gi_v.at[0]],
                                rows_v[t][b].at[pl.ds(r * 128, 128)],
                                sem_g[t][b]).wait()
                        for r in range(rpc):
                            pltpu.async_copy(
                                rows_v[t][b].at[pl.ds(r * 128, 128)],
                                acc_sh[t].at[si_v.at[i * rpc + r]],
                                sem_s[t][b], add=True)
                return carry

            lax.fori_loop(0, nch // nbuf, pair, 0)
            for b in range(nbuf):
                for t in range(nt):
                    for r in range(rpc):
                        pltpu.make_async_copy(
                            rows_v[t][b].at[pl.ds(r * 128, 128)],
                            acc_sh[t].at[si_v.at[0]],
                            sem_s[t][b]).wait()
            plsc.subcore_barrier()
            out_row = (n_phase * c + p) * rows_per_sc + s * stripe
            for t in range(nt):
                pltpu.sync_copy(
                    acc_sh[t].at[pl.ds(s * stripe, stripe)],
                    out_hbm[t].at[pl.ds(out_row, stripe)],
                )
            plsc.subcore_barrier()

    return k(gather_idx, scatter_idx, *tables, zeros)


def _tc_pre(xf, dis, W, b, Wr, br):
    def body(x_ref, d_ref, w_ref, b_ref, wr_ref, br_ref, hs_ref, y0_ref):
        x = x_ref[...]
        dis = jnp.max(d_ref[...], axis=-1, keepdims=True)
        h = jnp.dot(x, w_ref[...], preferred_element_type=jnp.float32) + b_ref[...]
        hs_ref[...] = dis * h
        y0_ref[...] = (
            jnp.dot(x, wr_ref[...], preferred_element_type=jnp.float32) + br_ref[...]
        )

    R = 2048
    grid = (BN // R,)
    blk = lambda r, cdim: pl.BlockSpec((r, cdim), lambda i: (i, 0))
    full = lambda a: pl.BlockSpec(a.shape, lambda i: (0,) * a.ndim)
    return pl.pallas_call(
        body,
        grid=grid,
        in_specs=[blk(R, 16), blk(R, 16), full(W), full(b), full(Wr), full(br)],
        out_specs=[blk(R, 64), blk(R, 64)],
        out_shape=[
            jax.ShapeDtypeStruct((BN, 64), jnp.float32),
            jax.ShapeDtypeStruct((BN, 64), jnp.float32),
        ],
    )(xf, dis, W, b, Wr, br)


def _tc_mid(acc, y0, dis, Wa, ba, Wec, bec):
    def body(a_ref, y_ref, d_ref, wa_ref, ba_ref, we_ref, be_ref,
             slo_ref, shi_ref, f_ref):
        dis = jnp.max(d_ref[...], axis=-1, keepdims=True)
        h1 = y_ref[...] + dis * a_ref[...]
        lg = jnp.dot(h1, wa_ref[...], preferred_element_type=jnp.float32) + ba_ref[...]
        lg = lg - jnp.max(lg, axis=-1, keepdims=True)
        ex = jnp.exp(lg)
        S = ex / jnp.sum(ex, axis=-1, keepdims=True)
        slo_ref[...] = S[:, :128]
        shi_ref[...] = S[:, 128:]
        f_ref[...] = (
            jnp.dot(h1, we_ref[...], preferred_element_type=jnp.float32) + be_ref[...]
        )

    R = 2048
    grid = (BN // R,)
    blk = lambda r, cdim: pl.BlockSpec((r, cdim), lambda i: (i, 0))
    full = lambda a: pl.BlockSpec(a.shape, lambda i: (0,) * a.ndim)
    return pl.pallas_call(
        body,
        grid=grid,
        in_specs=[blk(R, 64), blk(R, 64), blk(R, 16), full(Wa), full(ba),
                  full(Wec), full(bec)],
        out_specs=[blk(R, 128), blk(R, 128), blk(R, 64)],
        out_shape=[
            jax.ShapeDtypeStruct((BN, 128), jnp.float32),
            jax.ShapeDtypeStruct((BN, 128), jnp.float32),
            jax.ShapeDtypeStruct((BN, 64), jnp.float32),
        ],
    )(acc, y0, dis, Wa, ba, Wec, bec)


def _top4(A):
    M = A.shape[-1]
    lane = lax.broadcasted_iota(jnp.int32, A.shape, 1)
    cur = A
    idxs = []
    for _ in range(4):
        m = jnp.max(cur, axis=-1, keepdims=True)
        j = jnp.min(jnp.where(cur == m, lane, M), axis=-1, keepdims=True)
        idxs.append(j)
        cur = jnp.where(lane == j, _NEG, cur)
    return idxs


def _adj(idxs, M):
    lane = lax.broadcasted_iota(jnp.int32, (idxs[0].shape[0], M), 1)
    a = jnp.zeros((idxs[0].shape[0], M), jnp.float32)
    for j in idxs:
        a = a + (lane == j).astype(jnp.float32)
    return a


def _ct(a, b):
    return lax.dot_general(a, b, (((0,), (0,)), ((), ())),
                           preferred_element_type=jnp.float32)


def _gcn_small(x, Adj, W, b, Wr, br):
    h = jnp.dot(x, W, preferred_element_type=jnp.float32) + b
    y = jnp.dot(x, Wr, preferred_element_type=jnp.float32) + br
    return y + 0.25 * _ct(Adj, h)


def _pool_small(x, Adj, Wa, ba, Wec, bec, M, need_topk=True):
    lg = jnp.dot(x, Wa, preferred_element_type=jnp.float32) + ba
    lg = lg - jnp.max(lg, axis=-1, keepdims=True)
    ex = jnp.exp(lg)
    S = ex / jnp.sum(ex, axis=-1, keepdims=True)
    fcat = jnp.dot(x, Wec, preferred_element_type=jnp.float32) + bec
    Z = _ct(S, fcat)
    w = jnp.maximum(jnp.sum(S, axis=0, keepdims=True), 1e-10)
    col = lax.broadcasted_iota(jnp.int32, Z.shape, 1)
    xn = jnp.where(col < 3, Z / w.reshape(-1, 1), Z)
    if not need_topk:
        return xn, None
    A = _ct(S, jnp.dot(Adj, S, preferred_element_type=jnp.float32))
    return xn, _top4(A)


def _tc_tail(Slo, Shi, ASlo, AShi, fcat, TW):
    def body(slo_ref, shi_ref, alo_ref, ahi_ref, f_ref, *refs):
        w_refs = refs[:-2]
        xr_ref, z_ref = refs[-2:]
        P = {name: r[...] for name, r in zip(_TAIL_NAMES, w_refs)}
        Sg = jnp.concatenate([slo_ref[0], shi_ref[0]], axis=1)
        ASg = jnp.concatenate([alo_ref[0], ahi_ref[0]], axis=1)
        Z = _ct(Sg, f_ref[0])
        w = jnp.maximum(jnp.sum(Sg, axis=0, keepdims=True), 1e-10)
        col = lax.broadcasted_iota(jnp.int32, Z.shape, 1)
        x = jnp.where(col < 3, Z / w.reshape(-1, 1), Z)
        A = _ct(Sg, ASg)
        idxs = _top4(A)
        M = M1
        specs = [("enc_gcn2_", "enc_pool2_", 64), ("enc_gcn3_", "enc_pool3_", 16),
                 ("dec_gcn1_", "dec_pool1_", 64), ("dec_gcn2_", "dec_pool2_", 256),
                 ("dec_gcn3_", "dec_pool3_", 1024)]
        for i, (g, p, Mn) in enumerate(specs):
            Adj = _adj(idxs, M)
            x = _gcn_small(x, Adj, P[g + "W"], P[g + "b"], P[g + "Wr"],
                           P[g + "br"])
            last = i == len(specs) - 1
            x, idxs = _pool_small(x, Adj, P[p + "Wa"], P[p + "ba"],
                                  P[p + "Wec"], P[p + "bec"], Mn,
                                  need_topk=not last)
            M = Mn
            if p == "enc_pool3_":
                z_ref[0] = x
        hh = jnp.dot(x, P["head_W"], preferred_element_type=jnp.float32) + P["head_b"]
        lane = lax.broadcasted_iota(jnp.int32, hh.shape, 1)
        sig = 1.0 / (1.0 + jnp.exp(-hh))
        xr = jnp.where((lane >= 3) & (lane < 6), sig, hh)
        xr_ref[0] = xr[:1000]

    gblk = lambda shp: pl.BlockSpec((1,) + shp, lambda i: (i, 0, 0))
    full = lambda a: pl.BlockSpec(a.shape, lambda i: (0,) * a.ndim)
    return pl.pallas_call(
        body,
        grid=(B,),
        in_specs=[gblk((N, 128)), gblk((N, 128)), gblk((N, 128)),
                  gblk((N, 128)), gblk((N, 64))]
        + [full(TW[n]) for n in _TAIL_NAMES],
        out_specs=[gblk((1000, 16)), gblk((16, 256))],
        out_shape=[
            jax.ShapeDtypeStruct((B, 1000, 16), jnp.float32),
            jax.ShapeDtypeStruct((B, 16, 256), jnp.float32),
        ],
    )(Slo.reshape(B, N, 128), Shi.reshape(B, N, 128),
      ASlo.reshape(B, N, 128), AShi.reshape(B, N, 128),
      fcat.reshape(B, N, 64), *[TW[n] for n in _TAIL_NAMES])


_TAIL_NAMES = []
for _g, _p in [("enc_gcn2_", "enc_pool2_"), ("enc_gcn3_", "enc_pool3_"),
               ("dec_gcn1_", "dec_pool1_"), ("dec_gcn2_", "dec_pool2_"),
               ("dec_gcn3_", "dec_pool3_")]:
    _TAIL_NAMES += [_g + "W", _g + "b", _g + "Wr", _g + "br",
                    _p + "Wa", _p + "ba", _p + "Wec", _p + "bec"]
_TAIL_NAMES += ["head_W", "head_b"]


def _wecat(P, pre, Cin):
    Wec = jnp.concatenate([jnp.eye(Cin, dtype=jnp.float32)[:, :3],
                           P[pre + "We"]], axis=1)
    bec = jnp.concatenate([jnp.zeros(3, jnp.float32), P[pre + "be"]])
    return Wec, bec.reshape(1, -1)


def kernel(x, e_, params):
    P = params
    xf = x.reshape(BN, C)
    e_ = e_.astype(jnp.int32)
    row, col = e_[0], e_[1]
    goff = jnp.arange(B, dtype=jnp.int32)[:, None]
    half = (goff % 2) * N
    row_sc = (row[None] + half).reshape(-1, 128)
    row_g = (row[None] + goff * N).reshape(-1, 128)
    col_loc = jnp.tile(col, B).reshape(-1, 128)
    col_g = (col[None] + goff * N).reshape(-1, 128)
    row_loc = jnp.tile(row, B).reshape(-1, 128)

    zeros256 = jnp.zeros((256, 128), jnp.float32)
    zeros256_64 = jnp.zeros((256, 64), jnp.float32)

    deg = _sc_deg(row_sc).reshape(BN, 16)
    dis = jnp.where(deg > 0, deg ** -0.5, 0.0)

    hs, y0 = _tc_pre(xf, dis, P["enc_gcn1_W"], P["enc_gcn1_b"].reshape(1, -1),
                     P["enc_gcn1_Wr"], P["enc_gcn1_br"].reshape(1, -1))

    (acc,) = _sc_edge_accum(row_g, col_loc, [hs], zeros256_64, rows_per_sc=N,
                            chunks_per_tile=8, n_phase=2, w=64)

    Wec1, bec1 = _wecat(P, "enc_pool1_", 64)
    Slo, Shi, fcat = _tc_mid(acc, y0, dis, P["enc_pool1_Wa"],
                             P["enc_pool1_ba"].reshape(1, -1), Wec1, bec1)

    (ASlo,) = _sc_edge_accum(col_g, row_loc, [Slo], zeros256,
                             rows_per_sc=N, chunks_per_tile=8, n_phase=2)
    (AShi,) = _sc_edge_accum(col_g, row_loc, [Shi], zeros256,
                             rows_per_sc=N, chunks_per_tile=8, n_phase=2)

    TW = {}
    for g, p in [("enc_gcn2_", "enc_pool2_"), ("enc_gcn3_", "enc_pool3_"),
                 ("dec_gcn1_", "dec_pool1_"), ("dec_gcn2_", "dec_pool2_"),
                 ("dec_gcn3_", "dec_pool3_")]:
        TW[g + "W"] = P[g + "W"]
        TW[g + "b"] = P[g + "b"].reshape(1, -1)
        TW[g + "Wr"] = P[g + "Wr"]
        TW[g + "br"] = P[g + "br"].reshape(1, -1)
        if p == "dec_pool3_":
            Wa = jnp.pad(P[p + "Wa"], ((0, 0), (0, 24)))
            ba = jnp.pad(P[p + "ba"], (0, 24), constant_values=_NEG)
            TW[p + "Wa"], TW[p + "ba"] = Wa, ba.reshape(1, -1)
        else:
            TW[p + "Wa"] = P[p + "Wa"]
            TW[p + "ba"] = P[p + "ba"].reshape(1, -1)
        TW[p + "Wec"], TW[p + "bec"] = _wecat(P, p, P[g + "W"].shape[1])
    TW["head_W"] = P["head_W"]
    TW["head_b"] = P["head_b"].reshape(1, -1)

    xr, z = _tc_tail(Slo, Shi, ASlo, AShi, fcat, TW)
    return xr, z.reshape(B * 16, 256)
